# baseline (speedup 1.0000x reference)
import functools

import jax
import jax.numpy as jnp
from jax import lax
from jax.experimental import pallas as pl
from jax.experimental.pallas import tpu as pltpu
from jax.experimental.pallas import tpu_sc as plsc

DIM = 128
HM = 2
TAU = 0.1
MAX_DEGREE = 4
N_NODES = 10000
N_EDGES = 320000

NW = 32
EPT = N_EDGES // NW
CHUNK = 128
EPT_P = 10240
NCHUNK = EPT_P // CHUNK
ACC_ROWS = 10240
RPT = ACC_ROWS // 16
DUMMY_DST = 10200
DACC_ROWS = 10240
DRPT = DACC_ROWS // 16
DCH = 80
DNCH = EPT // DCH
DEG_W = 16

BLK = 1000
GRID = N_NODES // BLK


def _deg_partials(src_grp):
    mesh = plsc.VectorSubcoreMesh(core_axis_name="c", subcore_axis_name="s")

    @functools.partial(
        pl.kernel,
        out_type=jax.ShapeDtypeStruct((2, DACC_ROWS, DEG_W), jnp.float32),
        mesh=mesh,
        scratch_types=[
            pltpu.VMEM((DNCH, DCH), jnp.int32),
            pltpu.VMEM((DCH, DEG_W), jnp.float32),
            pltpu.VMEM((DCH, DEG_W), jnp.float32),
            pltpu.VMEM_SHARED((DACC_ROWS, DEG_W), jnp.float32),
        ],
    )
    def k(src_hbm, out_hbm, idx_v, ones_v, zero_v, acc):
        c = lax.axis_index("c")
        s = lax.axis_index("s")
        wid = s * 2 + c

        def fill(i, _):
            ones_v[i, :] = jnp.ones((DEG_W,), jnp.float32)
            zero_v[i, :] = jnp.zeros((DEG_W,), jnp.float32)
            return 0

        lax.fori_loop(0, DCH, fill, 0)

        def zacc(j, _):
            pltpu.sync_copy(zero_v, acc.at[pl.ds(s * DRPT + j * DCH, DCH)])
            return 0

        lax.fori_loop(0, DRPT // DCH, zacc, 0)
        plsc.subcore_barrier()

        pltpu.sync_copy(src_hbm.at[wid], idx_v)

        def step(j, _):
            pltpu.sync_copy(ones_v, acc.at[idx_v.at[j]], add=True)
            return 0

        lax.fori_loop(0, DNCH, step, 0)
        plsc.subcore_barrier()
        pltpu.sync_copy(acc.at[pl.ds(s * DRPT, DRPT)],
                        out_hbm.at[c, pl.ds(s * DRPT, DRPT)])

    return k(src_grp)


def _edge_partials(y, idx_pk):
    mesh = plsc.VectorSubcoreMesh(core_axis_name="c", subcore_axis_name="s")

    @functools.partial(
        pl.kernel,
        out_type=jax.ShapeDtypeStruct((2, ACC_ROWS, DIM), jnp.float32),
        mesh=mesh,
        scratch_types=[
            pltpu.VMEM((2, CHUNK), jnp.int32),
            pltpu.VMEM((2, CHUNK), jnp.int32),
            pltpu.VMEM((CHUNK, DIM), jnp.float32),
            pltpu.VMEM((CHUNK, DIM), jnp.float32),
            pltpu.VMEM_SHARED((ACC_ROWS, DIM), jnp.float32),
            pltpu.SemaphoreType.DMA,
            pltpu.SemaphoreType.DMA,
            pltpu.SemaphoreType.DMA,
            pltpu.SemaphoreType.DMA,
        ],
    )
    def k(y_hbm, pk_hbm, out_hbm, idx_a, idx_b, rows_a, rows_b, acc,
          sem_a, sem_b, sem_ia, sem_ib):
        c = lax.axis_index("c")
        s = lax.axis_index("s")
        wid = s * 2 + c

        def fill(i, _):
            def fill_lane(m, _):
                rows_a[i, pl.ds(m * 16, 16)] = jnp.zeros((16,), jnp.float32)
                return 0
            lax.fori_loop(0, DIM // 16, fill_lane, 0)
            return 0

        lax.fori_loop(0, CHUNK, fill, 0)

        def zacc(j, _):
            pltpu.sync_copy(rows_a, acc.at[pl.ds(s * RPT + j * CHUNK, CHUNK)])
            return 0

        lax.fori_loop(0, RPT // CHUNK, zacc, 0)
        plsc.subcore_barrier()

        def step(j, _):
            pltpu.sync_copy(pk_hbm.at[wid, j], idx_a)
            pltpu.async_copy(y_hbm.at[idx_a.at[0]], rows_a, sem_a).wait()
            pltpu.sync_copy(rows_a, acc.at[idx_a.at[1]], add=True)
            return 0

        lax.fori_loop(0, NCHUNK, step, 0)
        plsc.subcore_barrier()
        pltpu.sync_copy(acc.at[pl.ds(s * RPT, RPT)],
                        out_hbm.at[c, pl.ds(s * RPT, RPT)])

    return k(y, idx_pk)


def _node_spec():
    return pl.BlockSpec((BLK, DIM), lambda i: (i, 0))


def _deg_spec():
    return pl.BlockSpec((BLK, DEG_W), lambda i: (i, 0))


def _dis_block(d0_r, d1_r):
    return lax.rsqrt(d0_r[:, 0:1] + d1_r[:, 0:1])


def _prep(x, d0, d1):
    def body(x_r, d0_r, d1_r, y_r):
        y_r[...] = x_r[...] * _dis_block(d0_r, d1_r)

    return pl.pallas_call(
        body,
        grid=(GRID,),
        in_specs=[_node_spec(), _deg_spec(), _deg_spec()],
        out_specs=_node_spec(),
        out_shape=jax.ShapeDtypeStruct((N_NODES, DIM), jnp.float32),
    )(x, d0, d1)


def _update(k_step, curr, h, p0, p1, d0, d1):
    coef = -TAU / k_step

    def body(c_r, h_r, p0_r, p1_r, d0_r, d1_r, co_r, ho_r, yo_r):
        dis = _dis_block(d0_r, d1_r)
        agg = (p0_r[...] + p1_r[...]) * dis
        cn = coef * (c_r[...] - agg)
        co_r[...] = cn
        ho_r[...] = h_r[...] + cn
        yo_r[...] = cn * dis

    o = jax.ShapeDtypeStruct((N_NODES, DIM), jnp.float32)
    return pl.pallas_call(
        body,
        grid=(GRID,),
        in_specs=[_node_spec(), _node_spec(), _node_spec(), _node_spec(),
                  _deg_spec(), _deg_spec()],
        out_specs=[_node_spec(), _node_spec(), _node_spec()],
        out_shape=[o, o, o],
    )(curr, h, p0, p1, d0, d1)


def _final(x, curr, h, p0, p1, d0, d1, W1, b1r, W2, b2r):
    coef = -TAU / MAX_DEGREE

    def body(x_r, c_r, h_r, p0_r, p1_r, d0_r, d1_r, w1_r, b1_r, w2_r, b2_r,
             o_r):
        dis = _dis_block(d0_r, d1_r)
        agg = (p0_r[...] + p1_r[...]) * dis
        cn = coef * (c_r[...] - agg)
        hn = h_r[...] + cn
        a = (jnp.dot(x_r[...], w1_r[0:DIM, :], precision=lax.Precision.HIGHEST,
                     preferred_element_type=jnp.float32)
             + jnp.dot(hn, w1_r[DIM:2 * DIM, :],
                       precision=lax.Precision.HIGHEST,
                       preferred_element_type=jnp.float32)
             + b1_r[...])
        g = jax.nn.gelu(a)
        o_r[...] = (jnp.dot(g, w2_r[...], precision=lax.Precision.HIGHEST,
                            preferred_element_type=jnp.float32)
                    + b2_r[...])

    hid = DIM * HM
    return pl.pallas_call(
        body,
        grid=(GRID,),
        in_specs=[_node_spec(), _node_spec(), _node_spec(), _node_spec(),
                  _node_spec(), _deg_spec(), _deg_spec(),
                  pl.BlockSpec((2 * DIM, hid), lambda i: (0, 0)),
                  pl.BlockSpec((1, hid), lambda i: (0, 0)),
                  pl.BlockSpec((hid, DIM), lambda i: (0, 0)),
                  pl.BlockSpec((1, DIM), lambda i: (0, 0))],
        out_specs=_node_spec(),
        out_shape=jax.ShapeDtypeStruct((N_NODES, DIM), jnp.float32),
    )(x, curr, h, p0, p1, d0, d1, W1, b1r, W2, b2r)


def kernel(x, node_rep, edge_index, W1, b1, W2, b2):
    del node_rep
    src_flat = edge_index[0].astype(jnp.int32)
    dst_flat = edge_index[1].astype(jnp.int32)
    pad = EPT_P - EPT
    src_p = jnp.concatenate(
        [src_flat.reshape(NW, EPT),
         jnp.zeros((NW, pad), jnp.int32)], axis=1).reshape(NW, NCHUNK, CHUNK)
    dst_p = jnp.concatenate(
        [dst_flat.reshape(NW, EPT),
         jnp.full((NW, pad), DUMMY_DST, jnp.int32)],
        axis=1).reshape(NW, NCHUNK, CHUNK)
    idx_pk = jnp.stack([src_p, dst_p], axis=2)
    b1r = b1.reshape(1, -1)
    b2r = b2.reshape(1, -1)

    degp = _deg_partials(src_flat.reshape(NW, DNCH, DCH))
    d0, d1 = degp[0], degp[1]

    y = _prep(x, d0, d1)
    curr = x
    h = x
    out = None
    for k_step in range(1, MAX_DEGREE + 1):
        p = _edge_partials(y, idx_pk)
        p0, p1 = p[0], p[1]
        if k_step < MAX_DEGREE:
            curr, h, y = _update(k_step, curr, h, p0, p1, d0, d1)
        else:
            out = _final(x, curr, h, p0, p1, d0, d1, W1, b1r, W2, b2r)
    return out


# speedup vs baseline: 1.0000x; 1.0000x over previous
import functools

import jax
import jax.numpy as jnp
from jax import lax
from jax.experimental import pallas as pl
from jax.experimental.pallas import tpu as pltpu
from jax.experimental.pallas import tpu_sc as plsc

DIM = 128
HM = 2
TAU = 0.1
MAX_DEGREE = 4
N_NODES = 10000
N_EDGES = 320000

NW = 32
EPT = N_EDGES // NW
CHUNK = 128
EPT_P = 10240
NCHUNK = EPT_P // CHUNK
ACC_ROWS = 10240
RPT = ACC_ROWS // 16
DUMMY_DST = 10200
DACC_ROWS = 10240
DRPT = DACC_ROWS // 16
DCH = 80
DNCH = EPT // DCH
DEG_W = 16

BLK = 1000
GRID = N_NODES // BLK


def _deg_partials(src_grp):
    mesh = plsc.VectorSubcoreMesh(core_axis_name="c", subcore_axis_name="s")

    @functools.partial(
        pl.kernel,
        out_type=jax.ShapeDtypeStruct((2, DACC_ROWS, DEG_W), jnp.float32),
        mesh=mesh,
        scratch_types=[
            pltpu.VMEM((DNCH, DCH), jnp.int32),
            pltpu.VMEM((DCH, DEG_W), jnp.float32),
            pltpu.VMEM((DCH, DEG_W), jnp.float32),
            pltpu.VMEM_SHARED((DACC_ROWS, DEG_W), jnp.float32),
        ],
    )
    def k(src_hbm, out_hbm, idx_v, ones_v, zero_v, acc):
        c = lax.axis_index("c")
        s = lax.axis_index("s")
        wid = s * 2 + c

        def fill(i, _):
            ones_v[i, :] = jnp.ones((DEG_W,), jnp.float32)
            zero_v[i, :] = jnp.zeros((DEG_W,), jnp.float32)
            return 0

        lax.fori_loop(0, DCH, fill, 0)

        def zacc(j, _):
            pltpu.sync_copy(zero_v, acc.at[pl.ds(s * DRPT + j * DCH, DCH)])
            return 0

        lax.fori_loop(0, DRPT // DCH, zacc, 0)
        plsc.subcore_barrier()

        pltpu.sync_copy(src_hbm.at[wid], idx_v)

        def step(j, _):
            pltpu.sync_copy(ones_v, acc.at[idx_v.at[j]], add=True)
            return 0

        lax.fori_loop(0, DNCH, step, 0)
        plsc.subcore_barrier()
        pltpu.sync_copy(acc.at[pl.ds(s * DRPT, DRPT)],
                        out_hbm.at[c, pl.ds(s * DRPT, DRPT)])

    return k(src_grp)


def _edge_partials(y, idx_pk):
    mesh = plsc.VectorSubcoreMesh(core_axis_name="c", subcore_axis_name="s")

    @functools.partial(
        pl.kernel,
        out_type=jax.ShapeDtypeStruct((2, ACC_ROWS, DIM), jnp.float32),
        mesh=mesh,
        scratch_types=[
            pltpu.VMEM((2, CHUNK), jnp.int32),
            pltpu.VMEM((2, CHUNK), jnp.int32),
            pltpu.VMEM((CHUNK, DIM), jnp.float32),
            pltpu.VMEM((CHUNK, DIM), jnp.float32),
            pltpu.VMEM_SHARED((ACC_ROWS, DIM), jnp.float32),
            pltpu.SemaphoreType.DMA,
            pltpu.SemaphoreType.DMA,
            pltpu.SemaphoreType.DMA,
            pltpu.SemaphoreType.DMA,
        ],
    )
    def k(y_hbm, pk_hbm, out_hbm, idx_a, idx_b, rows_a, rows_b, acc,
          sem_a, sem_b, sem_ia, sem_ib):
        c = lax.axis_index("c")
        s = lax.axis_index("s")
        wid = s * 2 + c

        def fill(i, _):
            def fill_lane(m, _):
                rows_a[i, pl.ds(m * 16, 16)] = jnp.zeros((16,), jnp.float32)
                return 0
            lax.fori_loop(0, DIM // 16, fill_lane, 0)
            return 0

        lax.fori_loop(0, CHUNK, fill, 0)

        def zacc(j, _):
            pltpu.sync_copy(rows_a, acc.at[pl.ds(s * RPT + j * CHUNK, CHUNK)])
            return 0

        lax.fori_loop(0, RPT // CHUNK, zacc, 0)
        plsc.subcore_barrier()

        def step(j, _):
            pltpu.sync_copy(pk_hbm.at[wid, j], idx_a)
            pltpu.async_copy(y_hbm.at[idx_a.at[0]], rows_a, sem_a).wait()
            pltpu.sync_copy(rows_a, acc.at[idx_a.at[1]], add=True)
            return 0

        lax.fori_loop(0, NCHUNK, step, 0)
        plsc.subcore_barrier()
        pltpu.sync_copy(acc.at[pl.ds(s * RPT, RPT)],
                        out_hbm.at[c, pl.ds(s * RPT, RPT)])

    return k(y, idx_pk)


def _node_spec():
    return pl.BlockSpec((BLK, DIM), lambda i: (i, 0))


def _deg_spec():
    return pl.BlockSpec((BLK, DEG_W), lambda i: (i, 0))


def _dis_block(d0_r, d1_r):
    return lax.rsqrt(d0_r[:, 0:1] + d1_r[:, 0:1])


def _prep(x, d0, d1):
    def body(x_r, d0_r, d1_r, y_r):
        y_r[...] = x_r[...] * _dis_block(d0_r, d1_r)

    return pl.pallas_call(
        body,
        grid=(GRID,),
        in_specs=[_node_spec(), _deg_spec(), _deg_spec()],
        out_specs=_node_spec(),
        out_shape=jax.ShapeDtypeStruct((N_NODES, DIM), jnp.float32),
    )(x, d0, d1)


def _update(k_step, curr, h, p0, p1, d0, d1):
    coef = -TAU / k_step

    def body(c_r, h_r, p0_r, p1_r, d0_r, d1_r, co_r, ho_r, yo_r):
        dis = _dis_block(d0_r, d1_r)
        agg = (p0_r[...] + p1_r[...]) * dis
        cn = coef * (c_r[...] - agg)
        co_r[...] = cn
        ho_r[...] = h_r[...] + cn
        yo_r[...] = cn * dis

    o = jax.ShapeDtypeStruct((N_NODES, DIM), jnp.float32)
    return pl.pallas_call(
        body,
        grid=(GRID,),
        in_specs=[_node_spec(), _node_spec(), _node_spec(), _node_spec(),
                  _deg_spec(), _deg_spec()],
        out_specs=[_node_spec(), _node_spec(), _node_spec()],
        out_shape=[o, o, o],
    )(curr, h, p0, p1, d0, d1)


def _final(x, curr, h, p0, p1, d0, d1, W1, b1r, W2, b2r):
    coef = -TAU / MAX_DEGREE

    def body(x_r, c_r, h_r, p0_r, p1_r, d0_r, d1_r, w1_r, b1_r, w2_r, b2_r,
             o_r):
        dis = _dis_block(d0_r, d1_r)
        agg = (p0_r[...] + p1_r[...]) * dis
        cn = coef * (c_r[...] - agg)
        hn = h_r[...] + cn
        a = (jnp.dot(x_r[...], w1_r[0:DIM, :], precision=lax.Precision.HIGHEST,
                     preferred_element_type=jnp.float32)
             + jnp.dot(hn, w1_r[DIM:2 * DIM, :],
                       precision=lax.Precision.HIGHEST,
                       preferred_element_type=jnp.float32)
             + b1_r[...])
        g = jax.nn.gelu(a)
        o_r[...] = (jnp.dot(g, w2_r[...], precision=lax.Precision.HIGHEST,
                            preferred_element_type=jnp.float32)
                    + b2_r[...])

    hid = DIM * HM
    return pl.pallas_call(
        body,
        grid=(GRID,),
        in_specs=[_node_spec(), _node_spec(), _node_spec(), _node_spec(),
                  _node_spec(), _deg_spec(), _deg_spec(),
                  pl.BlockSpec((2 * DIM, hid), lambda i: (0, 0)),
                  pl.BlockSpec((1, hid), lambda i: (0, 0)),
                  pl.BlockSpec((hid, DIM), lambda i: (0, 0)),
                  pl.BlockSpec((1, DIM), lambda i: (0, 0))],
        out_specs=_node_spec(),
        out_shape=jax.ShapeDtypeStruct((N_NODES, DIM), jnp.float32),
    )(x, curr, h, p0, p1, d0, d1, W1, b1r, W2, b2r)


def kernel(x, node_rep, edge_index, W1, b1, W2, b2):
    del node_rep
    src_flat = edge_index[0].astype(jnp.int32)
    dst_flat = edge_index[1].astype(jnp.int32)
    pad = EPT_P - EPT
    src_p = jnp.concatenate(
        [src_flat.reshape(NW, EPT),
         jnp.zeros((NW, pad), jnp.int32)], axis=1).reshape(NW, NCHUNK, CHUNK)
    dummy_rows = (N_NODES + jnp.arange(NW, dtype=jnp.int32))[:, None]
    dst_p = jnp.concatenate(
        [dst_flat.reshape(NW, EPT),
         jnp.broadcast_to(dummy_rows, (NW, pad))],
        axis=1).reshape(NW, NCHUNK, CHUNK)
    idx_pk = jnp.stack([src_p, dst_p], axis=2)
    b1r = b1.reshape(1, -1)
    b2r = b2.reshape(1, -1)

    degp = _deg_partials(src_flat.reshape(NW, DNCH, DCH))
    d0, d1 = degp[0], degp[1]

    y = _prep(x, d0, d1)
    curr = x
    h = x
    out = None
    for k_step in range(1, MAX_DEGREE + 1):
        p = _edge_partials(y, idx_pk)
        p0, p1 = p[0], p[1]
        if k_step < MAX_DEGREE:
            curr, h, y = _update(k_step, curr, h, p0, p1, d0, d1)
        else:
            out = _final(x, curr, h, p0, p1, d0, d1, W1, b1r, W2, b2r)
    return out


# speedup vs baseline: 1.0667x; 1.0667x over previous
import functools

import jax
import jax.numpy as jnp
from jax import lax
from jax.experimental import pallas as pl
from jax.experimental.pallas import tpu as pltpu
from jax.experimental.pallas import tpu_sc as plsc

DIM = 128
HM = 2
TAU = 0.1
MAX_DEGREE = 4
N_NODES = 10000
N_EDGES = 320000

NW = 32
EPT = N_EDGES // NW
CHUNK = 128
EPT_P = 10240
NCHUNK = EPT_P // CHUNK
ACC_ROWS = 10240
RPT = ACC_ROWS // 16
DUMMY_DST = 10200
DACC_ROWS = 10240
DRPT = DACC_ROWS // 16
DCH = 80
DNCH = EPT // DCH
DEG_W = 16

BLK = 1000
GRID = N_NODES // BLK


def _deg_partials(src_grp):
    mesh = plsc.VectorSubcoreMesh(core_axis_name="c", subcore_axis_name="s")

    @functools.partial(
        pl.kernel,
        out_type=jax.ShapeDtypeStruct((2, DACC_ROWS, DEG_W), jnp.float32),
        mesh=mesh,
        scratch_types=[
            pltpu.VMEM((DNCH, DCH), jnp.int32),
            pltpu.VMEM((DCH, DEG_W), jnp.float32),
            pltpu.VMEM((DCH, DEG_W), jnp.float32),
            pltpu.VMEM_SHARED((DACC_ROWS, DEG_W), jnp.float32),
        ],
    )
    def k(src_hbm, out_hbm, idx_v, ones_v, zero_v, acc):
        c = lax.axis_index("c")
        s = lax.axis_index("s")
        wid = s * 2 + c

        def fill(i, _):
            ones_v[i, :] = jnp.ones((DEG_W,), jnp.float32)
            zero_v[i, :] = jnp.zeros((DEG_W,), jnp.float32)
            return 0

        lax.fori_loop(0, DCH, fill, 0)

        def zacc(j, _):
            pltpu.sync_copy(zero_v, acc.at[pl.ds(s * DRPT + j * DCH, DCH)])
            return 0

        lax.fori_loop(0, DRPT // DCH, zacc, 0)
        plsc.subcore_barrier()

        pltpu.sync_copy(src_hbm.at[wid], idx_v)

        def step(j, _):
            pltpu.sync_copy(ones_v, acc.at[idx_v.at[j]], add=True)
            return 0

        lax.fori_loop(0, DNCH, step, 0)
        plsc.subcore_barrier()
        pltpu.sync_copy(acc.at[pl.ds(s * DRPT, DRPT)],
                        out_hbm.at[c, pl.ds(s * DRPT, DRPT)])

    return k(src_grp)


def _edge_partials(y, idx_pk):
    mesh = plsc.VectorSubcoreMesh(core_axis_name="c", subcore_axis_name="s")

    @functools.partial(
        pl.kernel,
        out_type=jax.ShapeDtypeStruct((2, ACC_ROWS, DIM), jnp.float32),
        mesh=mesh,
        scratch_types=[
            pltpu.VMEM((NCHUNK, 2, CHUNK), jnp.int32),
            pltpu.VMEM((CHUNK, DIM), jnp.float32),
            pltpu.VMEM_SHARED((ACC_ROWS, DIM), jnp.float32),
            pltpu.SemaphoreType.DMA,
        ],
    )
    def k(y_hbm, pk_hbm, out_hbm, idx_v, rows_a, acc, sem_a):
        c = lax.axis_index("c")
        s = lax.axis_index("s")
        wid = s * 2 + c

        def fill(i, _):
            def fill_lane(m, _):
                rows_a[i, pl.ds(m * 16, 16)] = jnp.zeros((16,), jnp.float32)
                return 0
            lax.fori_loop(0, DIM // 16, fill_lane, 0)
            return 0

        lax.fori_loop(0, CHUNK, fill, 0)

        def zacc(j, _):
            pltpu.sync_copy(rows_a, acc.at[pl.ds(s * RPT + j * CHUNK, CHUNK)])
            return 0

        lax.fori_loop(0, RPT // CHUNK, zacc, 0)
        plsc.subcore_barrier()

        pltpu.sync_copy(pk_hbm.at[wid], idx_v)

        def step(j, _):
            pltpu.async_copy(y_hbm.at[idx_v.at[j, 0]], rows_a, sem_a).wait()
            pltpu.sync_copy(rows_a, acc.at[idx_v.at[j, 1]], add=True)
            return 0

        lax.fori_loop(0, NCHUNK, step, 0)
        plsc.subcore_barrier()
        pltpu.sync_copy(acc.at[pl.ds(s * RPT, RPT)],
                        out_hbm.at[c, pl.ds(s * RPT, RPT)])

    return k(y, idx_pk)


def _node_spec():
    return pl.BlockSpec((BLK, DIM), lambda i: (i, 0))


def _deg_spec():
    return pl.BlockSpec((BLK, DEG_W), lambda i: (i, 0))


def _dis_block(d0_r, d1_r):
    return lax.rsqrt(d0_r[:, 0:1] + d1_r[:, 0:1])


def _prep(x, d0, d1):
    def body(x_r, d0_r, d1_r, y_r):
        y_r[...] = x_r[...] * _dis_block(d0_r, d1_r)

    return pl.pallas_call(
        body,
        grid=(GRID,),
        in_specs=[_node_spec(), _deg_spec(), _deg_spec()],
        out_specs=_node_spec(),
        out_shape=jax.ShapeDtypeStruct((N_NODES, DIM), jnp.float32),
    )(x, d0, d1)


def _update(k_step, curr, h, p0, p1, d0, d1):
    coef = -TAU / k_step

    def body(c_r, h_r, p0_r, p1_r, d0_r, d1_r, co_r, ho_r, yo_r):
        dis = _dis_block(d0_r, d1_r)
        agg = (p0_r[...] + p1_r[...]) * dis
        cn = coef * (c_r[...] - agg)
        co_r[...] = cn
        ho_r[...] = h_r[...] + cn
        yo_r[...] = cn * dis

    o = jax.ShapeDtypeStruct((N_NODES, DIM), jnp.float32)
    return pl.pallas_call(
        body,
        grid=(GRID,),
        in_specs=[_node_spec(), _node_spec(), _node_spec(), _node_spec(),
                  _deg_spec(), _deg_spec()],
        out_specs=[_node_spec(), _node_spec(), _node_spec()],
        out_shape=[o, o, o],
    )(curr, h, p0, p1, d0, d1)


def _final(x, curr, h, p0, p1, d0, d1, W1, b1r, W2, b2r):
    coef = -TAU / MAX_DEGREE

    def body(x_r, c_r, h_r, p0_r, p1_r, d0_r, d1_r, w1_r, b1_r, w2_r, b2_r,
             o_r):
        dis = _dis_block(d0_r, d1_r)
        agg = (p0_r[...] + p1_r[...]) * dis
        cn = coef * (c_r[...] - agg)
        hn = h_r[...] + cn
        a = (jnp.dot(x_r[...], w1_r[0:DIM, :], precision=lax.Precision.HIGHEST,
                     preferred_element_type=jnp.float32)
             + jnp.dot(hn, w1_r[DIM:2 * DIM, :],
                       precision=lax.Precision.HIGHEST,
                       preferred_element_type=jnp.float32)
             + b1_r[...])
        g = jax.nn.gelu(a)
        o_r[...] = (jnp.dot(g, w2_r[...], precision=lax.Precision.HIGHEST,
                            preferred_element_type=jnp.float32)
                    + b2_r[...])

    hid = DIM * HM
    return pl.pallas_call(
        body,
        grid=(GRID,),
        in_specs=[_node_spec(), _node_spec(), _node_spec(), _node_spec(),
                  _node_spec(), _deg_spec(), _deg_spec(),
                  pl.BlockSpec((2 * DIM, hid), lambda i: (0, 0)),
                  pl.BlockSpec((1, hid), lambda i: (0, 0)),
                  pl.BlockSpec((hid, DIM), lambda i: (0, 0)),
                  pl.BlockSpec((1, DIM), lambda i: (0, 0))],
        out_specs=_node_spec(),
        out_shape=jax.ShapeDtypeStruct((N_NODES, DIM), jnp.float32),
    )(x, curr, h, p0, p1, d0, d1, W1, b1r, W2, b2r)


def kernel(x, node_rep, edge_index, W1, b1, W2, b2):
    del node_rep
    src_flat = edge_index[0].astype(jnp.int32)
    dst_flat = edge_index[1].astype(jnp.int32)
    pad = EPT_P - EPT
    src_p = jnp.concatenate(
        [src_flat.reshape(NW, EPT),
         jnp.zeros((NW, pad), jnp.int32)], axis=1).reshape(NW, NCHUNK, CHUNK)
    dummy_rows = (N_NODES + jnp.arange(NW, dtype=jnp.int32))[:, None]
    dst_p = jnp.concatenate(
        [dst_flat.reshape(NW, EPT),
         jnp.broadcast_to(dummy_rows, (NW, pad))],
        axis=1).reshape(NW, NCHUNK, CHUNK)
    idx_pk = jnp.stack([src_p, dst_p], axis=2)
    b1r = b1.reshape(1, -1)
    b2r = b2.reshape(1, -1)

    degp = _deg_partials(src_flat.reshape(NW, DNCH, DCH))
    d0, d1 = degp[0], degp[1]

    y = _prep(x, d0, d1)
    curr = x
    h = x
    out = None
    for k_step in range(1, MAX_DEGREE + 1):
        p = _edge_partials(y, idx_pk)
        p0, p1 = p[0], p[1]
        if k_step < MAX_DEGREE:
            curr, h, y = _update(k_step, curr, h, p0, p1, d0, d1)
        else:
            out = _final(x, curr, h, p0, p1, d0, d1, W1, b1r, W2, b2r)
    return out


# speedup vs baseline: 2.6476x; 2.4820x over previous
import functools

import jax
import jax.numpy as jnp
from jax import lax
from jax.experimental import pallas as pl
from jax.experimental.pallas import tpu as pltpu
from jax.experimental.pallas import tpu_sc as plsc

DIM = 128
HM = 2
TAU = 0.1
MAX_DEGREE = 4
N_NODES = 10000
N_EDGES = 320000

NW = 32
EPT = N_EDGES // NW
CHUNK = 80
NCHUNK = EPT // CHUNK
ACC_ROWS = 10240
RPT = ACC_ROWS // 16
DACC_ROWS = 10240
DRPT = DACC_ROWS // 16
DCH = 80
DNCH = EPT // DCH
DEG_W = 16

BLK = 1000
GRID = N_NODES // BLK


def _deg_partials(src_grp):
    mesh = plsc.VectorSubcoreMesh(core_axis_name="c", subcore_axis_name="s")

    @functools.partial(
        pl.kernel,
        out_type=jax.ShapeDtypeStruct((2, DACC_ROWS, DEG_W), jnp.float32),
        mesh=mesh,
        scratch_types=[
            pltpu.VMEM((DNCH, DCH), jnp.int32),
            pltpu.VMEM((DCH, DEG_W), jnp.float32),
            pltpu.VMEM((DCH, DEG_W), jnp.float32),
            pltpu.VMEM_SHARED((DACC_ROWS, DEG_W), jnp.float32),
        ],
    )
    def k(src_hbm, out_hbm, idx_v, ones_v, zero_v, acc):
        c = lax.axis_index("c")
        s = lax.axis_index("s")
        wid = s * 2 + c

        def fill(i, _):
            ones_v[i, :] = jnp.ones((DEG_W,), jnp.float32)
            zero_v[i, :] = jnp.zeros((DEG_W,), jnp.float32)
            return 0

        lax.fori_loop(0, DCH, fill, 0)

        def zacc(j, _):
            pltpu.sync_copy(zero_v, acc.at[pl.ds(s * DRPT + j * DCH, DCH)])
            return 0

        lax.fori_loop(0, DRPT // DCH, zacc, 0)
        plsc.subcore_barrier()

        pltpu.sync_copy(src_hbm.at[wid], idx_v)

        def step(j, _):
            pltpu.sync_copy(ones_v, acc.at[idx_v.at[j]], add=True)
            return 0

        lax.fori_loop(0, DNCH, step, 0)
        plsc.subcore_barrier()
        pltpu.sync_copy(acc.at[pl.ds(s * DRPT, DRPT)],
                        out_hbm.at[c, pl.ds(s * DRPT, DRPT)])

    return k(src_grp)


def _edge_partials(y, idx_pk):
    mesh = plsc.VectorSubcoreMesh(core_axis_name="c", subcore_axis_name="s")

    @functools.partial(
        pl.kernel,
        out_type=jax.ShapeDtypeStruct((2, ACC_ROWS, DIM), jnp.float32),
        mesh=mesh,
        scratch_types=[
            pltpu.VMEM((2, CHUNK), jnp.int32),
            pltpu.VMEM((2, CHUNK), jnp.int32),
            pltpu.VMEM((CHUNK, DIM), jnp.float32),
            pltpu.VMEM((CHUNK, DIM), jnp.float32),
            pltpu.VMEM_SHARED((ACC_ROWS, DIM), jnp.float32),
            pltpu.SemaphoreType.DMA,
            pltpu.SemaphoreType.DMA,
            pltpu.SemaphoreType.DMA,
            pltpu.SemaphoreType.DMA,
        ],
    )
    def k(y_hbm, pk_hbm, out_hbm, idx_a, idx_b, rows_a, rows_b, acc,
          sem_ga, sem_gb, sem_ia, sem_ib):
        c = lax.axis_index("c")
        s = lax.axis_index("s")
        wid = s * 2 + c

        def fill(i, _):
            def fill_lane(m, _):
                rows_a[i, pl.ds(m * 16, 16)] = jnp.zeros((16,), jnp.float32)
                return 0
            lax.fori_loop(0, DIM // 16, fill_lane, 0)
            return 0

        lax.fori_loop(0, CHUNK, fill, 0)

        def zacc(j, _):
            pltpu.sync_copy(rows_a, acc.at[pl.ds(s * RPT + j * CHUNK, CHUNK)])
            return 0

        lax.fori_loop(0, RPT // CHUNK, zacc, 0)
        plsc.subcore_barrier()

        pltpu.async_copy(pk_hbm.at[wid, 0], idx_a, sem_ia)
        pltpu.async_copy(pk_hbm.at[wid, 1], idx_b, sem_ib)
        pltpu.make_async_copy(pk_hbm.at[wid, 0], idx_a, sem_ia).wait()
        pltpu.async_copy(y_hbm.at[idx_a.at[0]], rows_a, sem_ga)

        def half(j, idx_x, idx_y, rows_x, rows_y, sem_gx, sem_gy, sem_ix,
                 sem_iy):
            pltpu.make_async_copy(y_hbm.at[idx_x.at[0]], rows_x, sem_gx).wait()

            @pl.when(j + 1 < NCHUNK)
            def _():
                pltpu.make_async_copy(pk_hbm.at[wid, 0], idx_y, sem_iy).wait()
                pltpu.async_copy(y_hbm.at[idx_y.at[0]], rows_y, sem_gy)

            pltpu.sync_copy(rows_x, acc.at[idx_x.at[1]], add=True)

            @pl.when(j + 2 < NCHUNK)
            def _():
                pltpu.async_copy(pk_hbm.at[wid, j + 2], idx_x, sem_ix)

        def step(j, _):
            @pl.when(j % 2 == 0)
            def _():
                half(j, idx_a, idx_b, rows_a, rows_b, sem_ga, sem_gb,
                     sem_ia, sem_ib)

            @pl.when(j % 2 == 1)
            def _():
                half(j, idx_b, idx_a, rows_b, rows_a, sem_gb, sem_ga,
                     sem_ib, sem_ia)

            return 0

        lax.fori_loop(0, NCHUNK, step, 0)
        plsc.subcore_barrier()
        pltpu.sync_copy(acc.at[pl.ds(s * RPT, RPT)],
                        out_hbm.at[c, pl.ds(s * RPT, RPT)])

    return k(y, idx_pk)


def _node_spec():
    return pl.BlockSpec((BLK, DIM), lambda i: (i, 0))


def _deg_spec():
    return pl.BlockSpec((BLK, DEG_W), lambda i: (i, 0))


def _dis_block(d0_r, d1_r):
    return lax.rsqrt(d0_r[:, 0:1] + d1_r[:, 0:1])


def _prep(x, d0, d1):
    def body(x_r, d0_r, d1_r, y_r):
        y_r[...] = x_r[...] * _dis_block(d0_r, d1_r)

    return pl.pallas_call(
        body,
        grid=(GRID,),
        in_specs=[_node_spec(), _deg_spec(), _deg_spec()],
        out_specs=_node_spec(),
        out_shape=jax.ShapeDtypeStruct((N_NODES, DIM), jnp.float32),
    )(x, d0, d1)


def _update(k_step, curr, h, p0, p1, d0, d1):
    coef = -TAU / k_step

    def body(c_r, h_r, p0_r, p1_r, d0_r, d1_r, co_r, ho_r, yo_r):
        dis = _dis_block(d0_r, d1_r)
        agg = (p0_r[...] + p1_r[...]) * dis
        cn = coef * (c_r[...] - agg)
        co_r[...] = cn
        ho_r[...] = h_r[...] + cn
        yo_r[...] = cn * dis

    o = jax.ShapeDtypeStruct((N_NODES, DIM), jnp.float32)
    return pl.pallas_call(
        body,
        grid=(GRID,),
        in_specs=[_node_spec(), _node_spec(), _node_spec(), _node_spec(),
                  _deg_spec(), _deg_spec()],
        out_specs=[_node_spec(), _node_spec(), _node_spec()],
        out_shape=[o, o, o],
    )(curr, h, p0, p1, d0, d1)


def _final(x, curr, h, p0, p1, d0, d1, W1, b1r, W2, b2r):
    coef = -TAU / MAX_DEGREE

    def body(x_r, c_r, h_r, p0_r, p1_r, d0_r, d1_r, w1_r, b1_r, w2_r, b2_r,
             o_r):
        dis = _dis_block(d0_r, d1_r)
        agg = (p0_r[...] + p1_r[...]) * dis
        cn = coef * (c_r[...] - agg)
        hn = h_r[...] + cn
        a = (jnp.dot(x_r[...], w1_r[0:DIM, :], precision=lax.Precision.HIGHEST,
                     preferred_element_type=jnp.float32)
             + jnp.dot(hn, w1_r[DIM:2 * DIM, :],
                       precision=lax.Precision.HIGHEST,
                       preferred_element_type=jnp.float32)
             + b1_r[...])
        g = jax.nn.gelu(a)
        o_r[...] = (jnp.dot(g, w2_r[...], precision=lax.Precision.HIGHEST,
                            preferred_element_type=jnp.float32)
                    + b2_r[...])

    hid = DIM * HM
    return pl.pallas_call(
        body,
        grid=(GRID,),
        in_specs=[_node_spec(), _node_spec(), _node_spec(), _node_spec(),
                  _node_spec(), _deg_spec(), _deg_spec(),
                  pl.BlockSpec((2 * DIM, hid), lambda i: (0, 0)),
                  pl.BlockSpec((1, hid), lambda i: (0, 0)),
                  pl.BlockSpec((hid, DIM), lambda i: (0, 0)),
                  pl.BlockSpec((1, DIM), lambda i: (0, 0))],
        out_specs=_node_spec(),
        out_shape=jax.ShapeDtypeStruct((N_NODES, DIM), jnp.float32),
    )(x, curr, h, p0, p1, d0, d1, W1, b1r, W2, b2r)


def kernel(x, node_rep, edge_index, W1, b1, W2, b2):
    del node_rep
    src_flat = edge_index[0].astype(jnp.int32)
    dst_flat = edge_index[1].astype(jnp.int32)
    src_p = src_flat.reshape(NW, NCHUNK, CHUNK)
    dst_p = dst_flat.reshape(NW, NCHUNK, CHUNK)
    idx_pk = jnp.stack([src_p, dst_p], axis=2)
    b1r = b1.reshape(1, -1)
    b2r = b2.reshape(1, -1)

    degp = _deg_partials(src_flat.reshape(NW, DNCH, DCH))
    d0, d1 = degp[0], degp[1]

    y = _prep(x, d0, d1)
    curr = x
    h = x
    out = None
    for k_step in range(1, MAX_DEGREE + 1):
        p = _edge_partials(y, idx_pk)
        p0, p1 = p[0], p[1]
        if k_step < MAX_DEGREE:
            curr, h, y = _update(k_step, curr, h, p0, p1, d0, d1)
        else:
            out = _final(x, curr, h, p0, p1, d0, d1, W1, b1r, W2, b2r)
    return out
